# baseline (speedup 1.0000x reference)
import functools

import jax
import jax.numpy as jnp
from jax import lax
from jax.experimental import pallas as pl
from jax.experimental.pallas import tpu as pltpu
from jax.experimental.pallas import tpu_sc as plsc

N_NODES = 100000
N_FEATS = 9
VOCAB = 100
HIDDEN = 512

NC = 2
NS = 16
NW = NC * NS
N_HSPLIT = 16
N_GSPLIT = NW // N_HSPLIT
HSLICE = HIDDEN // N_HSPLIT
HPAIR = HSLICE // 2
ROWS = N_FEATS * VOCAB
G_NODES = N_NODES // N_GSPLIT
CHUNK = 400
N_CHUNKS = G_NODES // CHUNK
TILES = CHUNK // 16


def _sc_body(idx_hbm, tab_hbm, out_hbm, table_sp, idx_v, gblk, stage_v, sems):
    c = lax.axis_index("c")
    s = lax.axis_index("s")
    wid = s * NC + c
    hid = wid % N_HSPLIT
    ng = wid // N_HSPLIT

    @pl.when(s == 0)
    def _():
        pltpu.sync_copy(tab_hbm, table_sp)

    plsc.subcore_barrier()

    hbase = hid * ROWS

    def start_gathers(t, b):
        nb = t * 16
        for f in range(N_FEATS):
            vis = idx_v[f, pl.ds(nb, 16)] + hbase
            pltpu.async_copy(table_sp.at[vis], gblk.at[b, f], sems.at[b])

    def wait_gathers(t, b):
        nb = t * 16
        for f in range(N_FEATS):
            vis = idx_v[f, pl.ds(nb, 16)] + hbase
            pltpu.make_async_copy(
                table_sp.at[vis], gblk.at[b, f], sems.at[b]).wait()

    def compute_tile(t, b):
        nb = t * 16
        for l in range(16):
            g = plsc.bitcast(gblk[b, 0, l], jnp.bfloat16)
            for f in range(1, N_FEATS):
                g = g + plsc.bitcast(gblk[b, f, l], jnp.bfloat16)
            lo, hi = plsc.unpack(g, format=plsc.PackFormat.INTERLEAVED,
                                 preferred_element_type=jnp.float32)
            stage_v[nb + l, pl.ds(0, 16)] = lo
            stage_v[nb + l, pl.ds(16, 16)] = hi

    def chunk_body(k, _):
        gbase = ng * G_NODES + k * CHUNK
        for f in range(N_FEATS):
            pltpu.sync_copy(
                idx_hbm.at[pl.ds(f * N_NODES + gbase, CHUNK)],
                idx_v.at[f])

        start_gathers(0, 0)

        def tile_body(i, _):
            b = lax.rem(i, 2)

            @pl.when(i < TILES - 1)
            def _():
                start_gathers(i + 1, 1 - b)

            wait_gathers(i, b)
            compute_tile(i, b)
            return 0

        lax.fori_loop(0, TILES, tile_body, 0)
        pltpu.sync_copy(
            stage_v,
            out_hbm.at[pl.ds(gbase, CHUNK), pl.ds(hid * HSLICE, HSLICE)])
        return 0

    lax.fori_loop(0, N_CHUNKS, chunk_body, 0)


@jax.jit
def kernel(x, tables):
    offs = (jnp.arange(N_FEATS, dtype=jnp.int32) * VOCAB)[None, :]
    idx_t = (x.astype(jnp.int32) + offs).T.reshape(-1)

    tb = tables.reshape(ROWS, HIDDEN).astype(jnp.bfloat16)
    tb = tb.reshape(ROWS, N_HSPLIT, 2, HPAIR).transpose(1, 0, 3, 2)
    tb_packed = lax.bitcast_convert_type(tb, jnp.int32).reshape(
        N_HSPLIT * ROWS, HPAIR)

    mesh = plsc.VectorSubcoreMesh(
        core_axis_name="c", subcore_axis_name="s",
        num_cores=NC, num_subcores=NS)
    f = pl.kernel(
        _sc_body,
        out_type=jax.ShapeDtypeStruct((N_NODES, HIDDEN), jnp.float32),
        mesh=mesh,
        scratch_types=[
            pltpu.VMEM_SHARED((N_HSPLIT * ROWS, HPAIR), jnp.int32),
            pltpu.VMEM((N_FEATS, CHUNK), jnp.int32),
            pltpu.VMEM((2, N_FEATS, 16, HPAIR), jnp.int32),
            pltpu.VMEM((CHUNK, HSLICE), jnp.float32),
            pltpu.SemaphoreType.DMA((2,)),
        ],
        compiler_params=pltpu.CompilerParams(
            use_tc_tiling_on_sc=False, needs_layout_passes=False),
    )
    return f(idx_t, tb_packed)


# speedup vs baseline: 1.0816x; 1.0816x over previous
import jax
import jax.numpy as jnp
from jax import lax
from jax.experimental import pallas as pl
from jax.experimental.pallas import tpu as pltpu
from jax.experimental.pallas import tpu_sc as plsc

N_NODES = 100000
N_FEATS = 9
VOCAB = 100
HIDDEN = 512

NC = 2
NS = 16
NW = NC * NS
N_HSPLIT = 16
N_GSPLIT = NW // N_HSPLIT
HSLICE = HIDDEN // N_HSPLIT
HPAIR = HSLICE // 2
ROWS = N_FEATS * VOCAB
G_NODES = N_NODES // N_GSPLIT
CHUNK = 400
N_CHUNKS = G_NODES // CHUNK
TILES = CHUNK // 16
ROWS_PAD = 904
TSIZE = HPAIR * ROWS_PAD


def _sc_body(idx_hbm, tab_hbm, out_hbm, table_v, idx_v, stage_v, isem, osem):
    c = lax.axis_index("c")
    s = lax.axis_index("s")
    wid = s * NC + c
    hid = wid % N_HSPLIT
    ng = wid // N_HSPLIT

    pltpu.sync_copy(tab_hbm.at[pl.ds(hid * TSIZE, TSIZE)], table_v)

    def out_slab(k):
        gbase = ng * G_NODES + k * CHUNK
        return out_hbm.at[pl.ds(gbase, CHUNK), pl.ds(hid * HSLICE, HSLICE)]

    def idx_fetch(k, b):
        gbase = ng * G_NODES + k * CHUNK
        pltpu.async_copy(
            idx_hbm.at[:, pl.ds(gbase, CHUNK)], idx_v.at[b], isem.at[b])

    def idx_wait(k, b):
        gbase = ng * G_NODES + k * CHUNK
        pltpu.make_async_copy(
            idx_hbm.at[:, pl.ds(gbase, CHUNK)], idx_v.at[b], isem.at[b]).wait()

    idx_fetch(0, 0)

    def chunk_body(k, _):
        b = lax.rem(k, 2)

        @pl.when(k < N_CHUNKS - 1)
        def _():
            idx_fetch(k + 1, 1 - b)

        idx_wait(k, b)

        @pl.when(k >= 2)
        def _():
            pltpu.make_async_copy(
                stage_v.at[b], out_slab(k - 2), osem.at[b]).wait()

        def tile(t, _):
            nb = t * 16
            rows = lax.iota(jnp.int32, 16) + nb
            vis = [idx_v[b, f, pl.ds(nb, 16)] for f in range(N_FEATS)]
            for j in range(HPAIR):
                ref_j = table_v.at[pl.ds(j * ROWS_PAD, ROWS_PAD)]
                g = plsc.bitcast(
                    plsc.load_gather(ref_j, [vis[0]]), jnp.bfloat16)
                for f in range(1, N_FEATS):
                    g = g + plsc.bitcast(
                        plsc.load_gather(ref_j, [vis[f]]), jnp.bfloat16)
                lo, hi = plsc.unpack(g, format=plsc.PackFormat.INTERLEAVED,
                                     preferred_element_type=jnp.float32)
                plsc.store_scatter(
                    stage_v.at[b],
                    [rows, jnp.full((16,), j, jnp.int32)], lo)
                plsc.store_scatter(
                    stage_v.at[b],
                    [rows, jnp.full((16,), HPAIR + j, jnp.int32)], hi)
            return 0

        lax.fori_loop(0, TILES, tile, 0)
        pltpu.async_copy(stage_v.at[b], out_slab(k), osem.at[b])
        return 0

    lax.fori_loop(0, N_CHUNKS, chunk_body, 0)
    for kk in (N_CHUNKS - 2, N_CHUNKS - 1):
        pltpu.make_async_copy(
            stage_v.at[kk % 2], out_slab(kk), osem.at[kk % 2]).wait()


@jax.jit
def kernel(x, tables):
    offs = (jnp.arange(N_FEATS, dtype=jnp.int32) * VOCAB)[None, :]
    idx_t = (x.astype(jnp.int32) + offs).T

    tb = tables.reshape(ROWS, HIDDEN).astype(jnp.bfloat16)
    tb = tb.reshape(ROWS, N_HSPLIT, 2, HPAIR).transpose(1, 3, 0, 2)
    tb_i32 = lax.bitcast_convert_type(tb, jnp.int32)
    tb_packed = jnp.pad(
        tb_i32, ((0, 0), (0, 0), (0, ROWS_PAD - ROWS))).reshape(-1)

    mesh = plsc.VectorSubcoreMesh(
        core_axis_name="c", subcore_axis_name="s",
        num_cores=NC, num_subcores=NS)
    f = pl.kernel(
        _sc_body,
        out_type=jax.ShapeDtypeStruct((N_NODES, HIDDEN), jnp.float32),
        mesh=mesh,
        scratch_types=[
            pltpu.VMEM((TSIZE,), jnp.int32),
            pltpu.VMEM((2, N_FEATS, CHUNK), jnp.int32),
            pltpu.VMEM((2, CHUNK, HSLICE), jnp.float32),
            pltpu.SemaphoreType.DMA((2,)),
            pltpu.SemaphoreType.DMA((2,)),
        ],
        compiler_params=pltpu.CompilerParams(
            use_tc_tiling_on_sc=False, needs_layout_passes=False),
    )
    return f(idx_t, tb_packed)


# speedup vs baseline: 2.0674x; 1.9115x over previous
import jax
import jax.numpy as jnp
from jax import lax
from jax.experimental import pallas as pl
from jax.experimental.pallas import tpu as pltpu
from jax.experimental.pallas import tpu_sc as plsc

N_NODES = 100000
N_FEATS = 9
VOCAB = 100
HIDDEN = 512

NC = 2
NS = 16
NW = NC * NS
N_HSPLIT = 16
N_GSPLIT = NW // N_HSPLIT
HSLICE = HIDDEN // N_HSPLIT
HPAIR = HSLICE // 2
ROWS = N_FEATS * VOCAB
G_NODES = N_NODES // N_GSPLIT
CHUNK = 400
N_CHUNKS = G_NODES // CHUNK
TILES = CHUNK // 16


def _sc_body(idx_hbm, tab_hbm, out_hbm, idx_v, acc_v, stage_v,
             isem, gsem, osem):
    c = lax.axis_index("c")
    s = lax.axis_index("s")
    wid = s * NC + c
    hid = wid % N_HSPLIT
    ng = wid // N_HSPLIT
    hbase = hid * ROWS

    zero32 = jnp.zeros((HSLICE,), jnp.bfloat16)

    def zero_body(i, _):
        for l in range(16):
            acc_v[0, i * 16 + l] = zero32
            acc_v[1, i * 16 + l] = zero32
        return 0
    lax.fori_loop(0, TILES, zero_body, 0)

    def out_slab(k):
        gbase = ng * G_NODES + k * CHUNK
        return out_hbm.at[pl.ds(gbase, CHUNK), pl.ds(hid * HSLICE, HSLICE)]

    def idx_fetch(k, b):
        gbase = ng * G_NODES + k * CHUNK
        pltpu.async_copy(
            idx_hbm.at[:, pl.ds(gbase, CHUNK)], idx_v.at[b], isem.at[b])

    def idx_wait(k, b):
        gbase = ng * G_NODES + k * CHUNK
        pltpu.make_async_copy(
            idx_hbm.at[:, pl.ds(gbase, CHUNK)], idx_v.at[b], isem.at[b]).wait()

    def adjust(b):
        def blk(i, _):
            for f in range(N_FEATS):
                v = idx_v[b, f, pl.ds(i * 16, 16)]
                idx_v[b, f, pl.ds(i * 16, 16)] = v + hbase
            return 0
        lax.fori_loop(0, TILES, blk, 0)

    def gathers_start(b):
        for f in range(N_FEATS):
            pltpu.async_copy(
                tab_hbm.at[idx_v.at[b, f]], acc_v.at[b], gsem.at[b],
                add=True)

    def gathers_wait(b):
        for f in range(N_FEATS):
            pltpu.make_async_copy(
                tab_hbm.at[idx_v.at[b, f]], acc_v.at[b], gsem.at[b]).wait()

    def unpack_chunk(b):
        def blk(i, _):
            nb = i * 16
            for l in range(16):
                g = acc_v[b, nb + l]
                lo, hi = plsc.unpack(g, format=plsc.PackFormat.INTERLEAVED,
                                     preferred_element_type=jnp.float32)
                stage_v[b, nb + l, pl.ds(0, HPAIR)] = lo
                stage_v[b, nb + l, pl.ds(HPAIR, HPAIR)] = hi
                acc_v[b, nb + l] = zero32
            return 0
        lax.fori_loop(0, TILES, blk, 0)

    idx_fetch(0, 0)
    idx_wait(0, 0)
    adjust(0)
    gathers_start(0)
    idx_fetch(1, 1)

    def chunk_body(k, _):
        b = lax.rem(k, 2)

        @pl.when(k < N_CHUNKS - 1)
        def _():
            idx_wait(k + 1, 1 - b)
            adjust(1 - b)
            gathers_start(1 - b)

        gathers_wait(b)

        @pl.when(k < N_CHUNKS - 2)
        def _():
            idx_fetch(k + 2, b)

        @pl.when(k >= 2)
        def _():
            pltpu.make_async_copy(
                stage_v.at[b], out_slab(k - 2), osem.at[b]).wait()

        unpack_chunk(b)
        pltpu.async_copy(stage_v.at[b], out_slab(k), osem.at[b])
        return 0

    lax.fori_loop(0, N_CHUNKS, chunk_body, 0)
    for kk in (N_CHUNKS - 2, N_CHUNKS - 1):
        pltpu.make_async_copy(
            stage_v.at[kk % 2], out_slab(kk), osem.at[kk % 2]).wait()


@jax.jit
def kernel(x, tables):
    offs = (jnp.arange(N_FEATS, dtype=jnp.int32) * VOCAB)[None, :]
    idx_t = (x.astype(jnp.int32) + offs).T

    tb = tables.reshape(ROWS, HIDDEN).astype(jnp.bfloat16)
    tb = tb.reshape(ROWS, N_HSPLIT, 2, HPAIR).transpose(1, 0, 3, 2)
    tb_rows = tb.reshape(N_HSPLIT * ROWS, HSLICE)

    mesh = plsc.VectorSubcoreMesh(
        core_axis_name="c", subcore_axis_name="s",
        num_cores=NC, num_subcores=NS)
    f = pl.kernel(
        _sc_body,
        out_type=jax.ShapeDtypeStruct((N_NODES, HIDDEN), jnp.float32),
        mesh=mesh,
        scratch_types=[
            pltpu.VMEM((2, N_FEATS, CHUNK), jnp.int32),
            pltpu.VMEM((2, CHUNK, HSLICE), jnp.bfloat16),
            pltpu.VMEM((2, CHUNK, HSLICE), jnp.float32),
            pltpu.SemaphoreType.DMA((2,)),
            pltpu.SemaphoreType.DMA((2,)),
            pltpu.SemaphoreType.DMA((2,)),
        ],
        compiler_params=pltpu.CompilerParams(
            use_tc_tiling_on_sc=False, needs_layout_passes=False),
    )
    return f(idx_t, tb_rows)


# speedup vs baseline: 2.2747x; 1.1003x over previous
import jax
import jax.numpy as jnp
from jax import lax
from jax.experimental import pallas as pl
from jax.experimental.pallas import tpu as pltpu
from jax.experimental.pallas import tpu_sc as plsc

N_NODES = 100000
N_FEATS = 9
VOCAB = 100
HIDDEN = 512

NC = 2
NS = 16
NW = NC * NS
N_HSPLIT = 8
N_GSPLIT = NW // N_HSPLIT
HSLICE = HIDDEN // N_HSPLIT
NQ = HSLICE // 32
ROWS = N_FEATS * VOCAB
G_NODES = N_NODES // N_GSPLIT
CHUNK = 200
N_CHUNKS = G_NODES // CHUNK
IDXPAD = 208
UNROLL = 4


def _sc_body(idx_hbm, tab_hbm, out_hbm, idx_v, acc_v, stage_v,
             isem, gsem, osem):
    c = lax.axis_index("c")
    s = lax.axis_index("s")
    wid = s * NC + c
    hid = wid % N_HSPLIT
    ng = wid // N_HSPLIT
    hbase = hid * ROWS

    zero32 = jnp.zeros((32,), jnp.bfloat16)

    def zero_body(i, _):
        for l in range(UNROLL):
            n = i * UNROLL + l
            for bb in range(2):
                for q in range(NQ):
                    acc_v[bb, n, pl.ds(q * 32, 32)] = zero32
        return 0
    lax.fori_loop(0, CHUNK // UNROLL, zero_body, 0)

    def out_slab(k):
        gbase = ng * G_NODES + k * CHUNK
        return out_hbm.at[pl.ds(gbase, CHUNK), pl.ds(hid * HSLICE, HSLICE)]

    def idx_fetch(k, b):
        gbase = ng * G_NODES + k * CHUNK
        pltpu.async_copy(
            idx_hbm.at[:, pl.ds(gbase, CHUNK)],
            idx_v.at[b, :, pl.ds(0, CHUNK)], isem.at[b])

    def idx_wait(k, b):
        gbase = ng * G_NODES + k * CHUNK
        pltpu.make_async_copy(
            idx_hbm.at[:, pl.ds(gbase, CHUNK)],
            idx_v.at[b, :, pl.ds(0, CHUNK)], isem.at[b]).wait()

    def adjust(b):
        def blk(i, _):
            for f in range(N_FEATS):
                v = idx_v[b, f, pl.ds(i * 16, 16)]
                idx_v[b, f, pl.ds(i * 16, 16)] = v + hbase
            return 0
        lax.fori_loop(0, IDXPAD // 16, blk, 0)

    def gathers_start(b):
        for f in range(N_FEATS):
            pltpu.async_copy(
                tab_hbm.at[idx_v.at[b, f, pl.ds(0, CHUNK)]], acc_v.at[b],
                gsem.at[b], add=True)

    def gathers_wait(b):
        for f in range(N_FEATS):
            pltpu.make_async_copy(
                tab_hbm.at[idx_v.at[b, f, pl.ds(0, CHUNK)]], acc_v.at[b],
                gsem.at[b]).wait()

    def unpack_chunk(b):
        def blk(i, _):
            for l in range(UNROLL):
                n = i * UNROLL + l
                for q in range(NQ):
                    g = acc_v[b, n, pl.ds(q * 32, 32)]
                    lo, hi = plsc.unpack(
                        g, format=plsc.PackFormat.INTERLEAVED,
                        preferred_element_type=jnp.float32)
                    stage_v[b, n, pl.ds(q * 32, 16)] = lo
                    stage_v[b, n, pl.ds(q * 32 + 16, 16)] = hi
                    acc_v[b, n, pl.ds(q * 32, 32)] = zero32
            return 0
        lax.fori_loop(0, CHUNK // UNROLL, blk, 0)

    idx_fetch(0, 0)
    idx_wait(0, 0)
    adjust(0)
    gathers_start(0)
    idx_fetch(1, 1)

    def chunk_body(k, _):
        b = lax.rem(k, 2)

        @pl.when(k < N_CHUNKS - 1)
        def _():
            idx_wait(k + 1, 1 - b)
            adjust(1 - b)
            gathers_start(1 - b)

        gathers_wait(b)

        @pl.when(k < N_CHUNKS - 2)
        def _():
            idx_fetch(k + 2, b)

        @pl.when(k >= 2)
        def _():
            pltpu.make_async_copy(
                stage_v.at[b], out_slab(k - 2), osem.at[b]).wait()

        unpack_chunk(b)
        pltpu.async_copy(stage_v.at[b], out_slab(k), osem.at[b])
        return 0

    lax.fori_loop(0, N_CHUNKS, chunk_body, 0)
    for kk in (N_CHUNKS - 2, N_CHUNKS - 1):
        pltpu.make_async_copy(
            stage_v.at[kk % 2], out_slab(kk), osem.at[kk % 2]).wait()


@jax.jit
def kernel(x, tables):
    offs = (jnp.arange(N_FEATS, dtype=jnp.int32) * VOCAB)[None, :]
    idx_t = (x.astype(jnp.int32) + offs).T

    tb = tables.reshape(ROWS, HIDDEN).astype(jnp.bfloat16)
    tb = tb.reshape(ROWS, N_HSPLIT, NQ, 2, 16).transpose(1, 0, 2, 4, 3)
    tb_rows = tb.reshape(N_HSPLIT * ROWS, HSLICE)

    mesh = plsc.VectorSubcoreMesh(
        core_axis_name="c", subcore_axis_name="s",
        num_cores=NC, num_subcores=NS)
    f = pl.kernel(
        _sc_body,
        out_type=jax.ShapeDtypeStruct((N_NODES, HIDDEN), jnp.float32),
        mesh=mesh,
        scratch_types=[
            pltpu.VMEM((2, N_FEATS, IDXPAD), jnp.int32),
            pltpu.VMEM((2, CHUNK, HSLICE), jnp.bfloat16),
            pltpu.VMEM((2, CHUNK, HSLICE), jnp.float32),
            pltpu.SemaphoreType.DMA((2,)),
            pltpu.SemaphoreType.DMA((2,)),
            pltpu.SemaphoreType.DMA((2,)),
        ],
        compiler_params=pltpu.CompilerParams(
            use_tc_tiling_on_sc=False, needs_layout_passes=False),
    )
    return f(idx_t, tb_rows)
